# Initial kernel scaffold; baseline (speedup 1.0000x reference)
import jax
import jax.numpy as jnp
from jax.experimental import pallas as pl


def kernel(x, row, col, xyz):
    raise NotImplementedError("write your pallas kernel here")


# speedup vs baseline: 34.8518x; 34.8518x over previous
import functools

import jax
import jax.numpy as jnp
from jax import lax
from jax.experimental import pallas as pl
from jax.experimental.pallas import tpu as pltpu
from jax.experimental.pallas import tpu_sc as plsc

_N = 100000
_E = 1600000
_F = 8
_NC = 2
_NS = 16
_NW = _NC * _NS
_EW = _E // _NW
_CHUNK = 2000
_RPS = 6256
_NPAD = _RPS * _NS


def _sc_segment_sums(table, row, col, zeros):
    mesh = plsc.VectorSubcoreMesh(core_axis_name="c", subcore_axis_name="s")

    @functools.partial(
        pl.kernel,
        out_type=jax.ShapeDtypeStruct((_NC, _NPAD, _F), jnp.float32),
        mesh=mesh,
        scratch_types=[
            pltpu.VMEM_SHARED((_NPAD, _F), jnp.float32),
            pltpu.VMEM((_CHUNK,), jnp.int32),
            pltpu.VMEM((_CHUNK,), jnp.int32),
            pltpu.VMEM((_CHUNK, _F), jnp.float32),
        ],
        compiler_params=pltpu.CompilerParams(use_tc_tiling_on_sc=False),
    )
    def kern(table_hbm, row_hbm, col_hbm, zeros_hbm, out_hbm, acc, rbuf, cbuf, vals):
        c = lax.axis_index("c")
        s = lax.axis_index("s")
        wid = c * _NS + s

        pltpu.sync_copy(zeros_hbm.at[pl.ds(s * _RPS, _RPS)],
                        acc.at[pl.ds(s * _RPS, _RPS)])
        plsc.subcore_barrier()

        @pl.loop(0, _EW, step=_CHUNK)
        def _(i):
            base = wid * _EW + i
            pltpu.sync_copy(col_hbm.at[pl.ds(base, _CHUNK)], cbuf)
            pltpu.sync_copy(row_hbm.at[pl.ds(base, _CHUNK)], rbuf)
            pltpu.sync_copy(table_hbm.at[cbuf], vals)
            pltpu.sync_copy(vals, acc.at[rbuf], add=True)

        plsc.subcore_barrier()
        pltpu.sync_copy(acc.at[pl.ds(s * _RPS, _RPS)],
                        out_hbm.at[c].at[pl.ds(s * _RPS, _RPS)])

    return kern(table, row, col, zeros)


def _tc_finish(pt, xt, zt):
    def body(p_ref, x_ref, z_ref, o_ref):
        p = p_ref[0] + p_ref[1]
        cnt = jnp.maximum(p[6:7, :], 1.0)
        mean = p[0:6, :] / cnt
        mx = mean[0:3, :]
        mz = mean[3:6, :]
        nx = jnp.sqrt(jnp.sum(mx * mx, axis=0, keepdims=True))
        nz = jnp.sqrt(jnp.sum(mz * mz, axis=0, keepdims=True))
        xd = jnp.abs(x_ref[...] * 100.0 - mx / nx * 100.0)
        zd = jnp.abs(z_ref[...] * 100.0 - mz / nz * 100.0)
        dif = jnp.abs(xd - zd)
        o_ref[0, 0] = jnp.sum(dif) * (1.0 / (_N * 3))

    out = pl.pallas_call(
        body,
        out_shape=jax.ShapeDtypeStruct((1, 1), jnp.float32),
        out_specs=pl.BlockSpec(memory_space=pltpu.SMEM),
    )(pt, xt, zt)
    return out[0, 0]


def kernel(x, row, col, xyz):
    table = jnp.concatenate(
        [x * 100.0, xyz * 100.0,
         jnp.ones((_N, 1), jnp.float32), jnp.zeros((_N, 1), jnp.float32)],
        axis=1)
    zeros = jnp.zeros((_NPAD, _F), jnp.float32)
    partials = _sc_segment_sums(table, row, col, zeros)
    pt = jnp.transpose(partials[:, :_N, :], (0, 2, 1))
    return _tc_finish(pt, x.T, xyz.T)


# speedup vs baseline: 55.5220x; 1.5931x over previous
import jax
import jax.numpy as jnp
from jax import lax
from jax.experimental import pallas as pl
from jax.experimental.pallas import tpu as pltpu
from jax.experimental.pallas import tpu_sc as plsc

_N = 100000
_E = 1600000
_F = 8
_G = 16
_NR = _N // _G
_NPAD = 100096
_NRP = _NPAD // _G
_NC = 2
_NS = 16
_NW = _NC * _NS
_EW = _E // _NW
_CHUNK = 2000
_ZRPS = _NPAD // _NS
_BFIN = 3128


def _hi_dot(a, b):
    return lax.dot_general(a, b, (((1,), (0,)), ((), ())),
                           precision=lax.Precision.HIGHEST,
                           preferred_element_type=jnp.float32)


def _tc_build_table(xt3, zt3):
    def body(x_ref, z_ref, o_ref):
        i16 = lax.broadcasted_iota(jnp.int32, (_G, 128), 0)
        l16 = lax.broadcasted_iota(jnp.int32, (_G, 128), 1)
        t = jnp.zeros((_BFIN, 128), jnp.float32)
        for f in range(3):
            ef = (l16 == 8 * i16 + f).astype(jnp.float32)
            t = t + _hi_dot(x_ref[f], ef)
            ez = (l16 == 8 * i16 + 3 + f).astype(jnp.float32)
            t = t + _hi_dot(z_ref[f], ez)
        lane = lax.broadcasted_iota(jnp.int32, (_BFIN, 128), 1)
        o_ref[...] = t + (lane % 8 == 6).astype(jnp.float32)

    return pl.pallas_call(
        body,
        grid=(_NRP // _BFIN,),
        in_specs=[pl.BlockSpec((3, _BFIN, _G), lambda i: (0, i, 0)),
                  pl.BlockSpec((3, _BFIN, _G), lambda i: (0, i, 0))],
        out_specs=pl.BlockSpec((_BFIN, 128), lambda i: (i, 0)),
        out_shape=jax.ShapeDtypeStruct((_NRP, 128), jnp.float32),
    )(xt3, zt3)


def _sc_segment_sums(tab, row, col, zeros):
    mesh = plsc.VectorSubcoreMesh(core_axis_name="c", subcore_axis_name="s")

    @pl.kernel(
        out_type=jax.ShapeDtypeStruct((_NC, _NPAD, _F), jnp.float32),
        mesh=mesh,
        scratch_types=[
            pltpu.VMEM_SHARED((_NPAD, _F), jnp.float32),
            pltpu.VMEM((_CHUNK,), jnp.int32),
            pltpu.VMEM((_CHUNK,), jnp.int32),
            pltpu.VMEM((_CHUNK, _F), jnp.float32),
        ],
        compiler_params=pltpu.CompilerParams(use_tc_tiling_on_sc=False),
    )
    def kern(tab_hbm, row_hbm, col_hbm, zeros_hbm, out_hbm, acc,
             rbuf, cbuf, vals):
        c = lax.axis_index("c")
        s = lax.axis_index("s")
        wid = c * _NS + s
        nbase = s * _ZRPS

        pltpu.sync_copy(zeros_hbm.at[pl.ds(nbase, _ZRPS)],
                        acc.at[pl.ds(nbase, _ZRPS)])
        plsc.subcore_barrier()

        @pl.loop(0, _EW, step=_CHUNK)
        def _(i):
            base = wid * _EW + i
            pltpu.sync_copy(col_hbm.at[pl.ds(base, _CHUNK)], cbuf)
            pltpu.sync_copy(row_hbm.at[pl.ds(base, _CHUNK)], rbuf)
            pltpu.sync_copy(tab_hbm.at[cbuf], vals)
            pltpu.sync_copy(vals, acc.at[rbuf], add=True)

        plsc.subcore_barrier()
        pltpu.sync_copy(acc.at[pl.ds(nbase, _ZRPS)],
                        out_hbm.at[c].at[pl.ds(nbase, _ZRPS)])

    return kern(tab, row, col, zeros)


def _tc_finish(partials, tabi):
    def body(p_ref, t_ref, o_ref):
        p = p_ref[0] + p_ref[1]
        t = t_ref[...]

        r = lax.broadcasted_iota(jnp.int32, (128, 128), 0)
        l = lax.broadcasted_iota(jnp.int32, (128, 128), 1)
        same_grp = (r // 8) == (l // 8)
        fr, fl = r % 8, l % 8
        mc = (same_grp & (fr == 6)).astype(jnp.float32)
        grp_r = jnp.where(fr < 3, 0, jnp.where(fr < 6, 1, 2))
        grp_l = jnp.where(fl < 3, 0, jnp.where(fl < 6, 1, 0))
        mn = (same_grp & (grp_r < 2) & (grp_r == grp_l)).astype(jnp.float32)
        ms = (same_grp & (fr == fl + 3) & (fl < 3)).astype(jnp.float32)

        cnt = jnp.maximum(_hi_dot(p, mc), 1.0)
        mean = p / cnt
        norm = _hi_dot(mean * mean, mn)
        dir_ = mean * jax.lax.rsqrt(norm)
        d1 = jnp.abs(t - dir_)
        zdx = _hi_dot(d1, ms)
        lane = lax.broadcasted_iota(jnp.int32, (_BFIN, 128), 1)
        rowi = (pl.program_id(0) * _BFIN
                + lax.broadcasted_iota(jnp.int32, (_BFIN, 128), 0))
        dif = jnp.where((lane % 8 < 3) & (rowi < _NR), jnp.abs(d1 - zdx), 0.0)
        part = jnp.sum(dif) * (100.0 / (_N * 3))

        @pl.when(pl.program_id(0) == 0)
        def _():
            o_ref[0, 0] = 0.0

        o_ref[0, 0] += part

    out = pl.pallas_call(
        body,
        grid=(_NRP // _BFIN,),
        in_specs=[pl.BlockSpec((_NC, _BFIN, 128), lambda i: (0, i, 0)),
                  pl.BlockSpec((_BFIN, 128), lambda i: (i, 0))],
        out_specs=pl.BlockSpec((1, 1), lambda i: (0, 0),
                               memory_space=pltpu.SMEM),
        out_shape=jax.ShapeDtypeStruct((1, 1), jnp.float32),
    )(partials, tabi)
    return out[0, 0]


def kernel(x, row, col, xyz):
    zeros = jnp.zeros((_NPAD, _F), jnp.float32)
    xt3 = jnp.reshape(jnp.pad(x.T, ((0, 0), (0, _NPAD - _N))), (3, _NRP, _G))
    zt3 = jnp.reshape(jnp.pad(xyz.T, ((0, 0), (0, _NPAD - _N))), (3, _NRP, _G))
    tabi = _tc_build_table(xt3, zt3)
    tab = jnp.reshape(tabi, (_NPAD, _F))
    partials = _sc_segment_sums(tab, row, col, zeros)
    pint = jnp.reshape(partials, (_NC, _NRP, 128))
    return _tc_finish(pint, tabi)


# speedup vs baseline: 62.2032x; 1.1203x over previous
import jax
import jax.numpy as jnp
from jax import lax
from jax.experimental import pallas as pl
from jax.experimental.pallas import tpu as pltpu
from jax.experimental.pallas import tpu_sc as plsc

_N = 100000
_E = 1600000
_F = 8
_G = 16
_NR = _N // _G
_NPAD = 100096
_NRP = _NPAD // _G
_NC = 2
_NS = 16
_NW = _NC * _NS
_EW = _E // _NW
_CHUNK = 2000
_ZRPS = _NPAD // _NS
_BFIN = 3128


def _hi_dot(a, b):
    return lax.dot_general(a, b, (((1,), (0,)), ((), ())),
                           precision=lax.Precision.HIGHEST,
                           preferred_element_type=jnp.float32)


def _tc_build_table(xt3, zt3):
    def body(x_ref, z_ref, o_ref):
        i16 = lax.broadcasted_iota(jnp.int32, (_G, 128), 0)
        l16 = lax.broadcasted_iota(jnp.int32, (_G, 128), 1)
        t = jnp.zeros((_BFIN, 128), jnp.float32)
        for f in range(3):
            ef = (l16 == 8 * i16 + f).astype(jnp.float32)
            t = t + _hi_dot(x_ref[f], ef)
            ez = (l16 == 8 * i16 + 3 + f).astype(jnp.float32)
            t = t + _hi_dot(z_ref[f], ez)
        lane = lax.broadcasted_iota(jnp.int32, (_BFIN, 128), 1)
        o_ref[...] = t + (lane % 8 == 6).astype(jnp.float32)

    return pl.pallas_call(
        body,
        grid=(_NRP // _BFIN,),
        in_specs=[pl.BlockSpec((3, _BFIN, _G), lambda i: (0, i, 0)),
                  pl.BlockSpec((3, _BFIN, _G), lambda i: (0, i, 0))],
        out_specs=pl.BlockSpec((_BFIN, 128), lambda i: (i, 0)),
        out_shape=jax.ShapeDtypeStruct((_NRP, 128), jnp.float32),
    )(xt3, zt3)


def _sc_segment_sums(tab, row, col, zeros):
    mesh = plsc.VectorSubcoreMesh(core_axis_name="c", subcore_axis_name="s")

    @pl.kernel(
        out_type=jax.ShapeDtypeStruct((_NC, _NPAD, _F), jnp.float32),
        mesh=mesh,
        scratch_types=[
            pltpu.VMEM_SHARED((_NPAD, _F), jnp.float32),
            pltpu.VMEM((_CHUNK,), jnp.int32),
            pltpu.VMEM((_CHUNK,), jnp.int32),
            pltpu.VMEM((_CHUNK, _F), jnp.float32),
        ],
        compiler_params=pltpu.CompilerParams(use_tc_tiling_on_sc=False),
    )
    def kern(tab_hbm, row_hbm, col_hbm, zeros_hbm, out_hbm, acc,
             rbuf, cbuf, vals):
        c = lax.axis_index("c")
        s = lax.axis_index("s")
        wid = c * _NS + s
        nbase = s * _ZRPS

        pltpu.sync_copy(zeros_hbm.at[pl.ds(nbase, _ZRPS)],
                        acc.at[pl.ds(nbase, _ZRPS)])
        plsc.subcore_barrier()

        @pl.loop(0, _EW, step=_CHUNK)
        def _(i):
            base = wid * _EW + i
            pltpu.sync_copy(col_hbm.at[pl.ds(base, _CHUNK)], cbuf)
            pltpu.sync_copy(row_hbm.at[pl.ds(base, _CHUNK)], rbuf)
            pltpu.sync_copy(tab_hbm.at[cbuf], vals)
            pltpu.sync_copy(vals, acc.at[rbuf], add=True)

        plsc.subcore_barrier()
        pltpu.sync_copy(acc.at[pl.ds(nbase, _ZRPS)],
                        out_hbm.at[c].at[pl.ds(nbase, _ZRPS)])

    return kern(tab, row, col, zeros)


def _tc_finish(partials, tabi):
    def body(p_ref, t_ref, o_ref):
        p = p_ref[0] + p_ref[1]
        t = t_ref[...]

        lane = lax.broadcasted_iota(jnp.int32, (_BFIN, 128), 1)
        f = lane % 8
        at0 = f == 0
        at3 = f == 3

        def grp_bcast(head):
            return head + pltpu.roll(head, 1, 1) + pltpu.roll(head, 2, 1)

        chead = jnp.where(at0, pltpu.roll(p, 122, 1),
                          jnp.where(at3, pltpu.roll(p, 125, 1), 0.0))
        cnt = jnp.maximum(grp_bcast(chead), 1.0)
        mean = p / cnt
        sq = mean * mean
        s3 = sq + pltpu.roll(sq, 127, 1) + pltpu.roll(sq, 126, 1)
        nhead = jnp.where(at0 | at3, s3, 0.0)
        dir_ = mean * jax.lax.rsqrt(grp_bcast(nhead))
        d1 = jnp.abs(t - dir_)
        zdx = pltpu.roll(d1, 125, 1)
        rowi = (pl.program_id(0) * _BFIN
                + lax.broadcasted_iota(jnp.int32, (_BFIN, 128), 0))
        dif = jnp.where((f < 3) & (rowi < _NR), jnp.abs(d1 - zdx), 0.0)
        part = jnp.sum(dif) * (100.0 / (_N * 3))

        @pl.when(pl.program_id(0) == 0)
        def _():
            o_ref[0, 0] = 0.0

        o_ref[0, 0] += part

    out = pl.pallas_call(
        body,
        grid=(_NRP // _BFIN,),
        in_specs=[pl.BlockSpec((_NC, _BFIN, 128), lambda i: (0, i, 0)),
                  pl.BlockSpec((_BFIN, 128), lambda i: (i, 0))],
        out_specs=pl.BlockSpec((1, 1), lambda i: (0, 0),
                               memory_space=pltpu.SMEM),
        out_shape=jax.ShapeDtypeStruct((1, 1), jnp.float32),
    )(partials, tabi)
    return out[0, 0]


def kernel(x, row, col, xyz):
    zeros = jnp.zeros((_NPAD, _F), jnp.float32)
    xt3 = jnp.reshape(jnp.pad(x.T, ((0, 0), (0, _NPAD - _N))), (3, _NRP, _G))
    zt3 = jnp.reshape(jnp.pad(xyz.T, ((0, 0), (0, _NPAD - _N))), (3, _NRP, _G))
    tabi = _tc_build_table(xt3, zt3)
    tab = jnp.reshape(tabi, (_NPAD, _F))
    partials = _sc_segment_sums(tab, row, col, zeros)
    pint = jnp.reshape(partials, (_NC, _NRP, 128))
    return _tc_finish(pint, tabi)


# speedup vs baseline: 69.3816x; 1.1154x over previous
import jax
import jax.numpy as jnp
from jax import lax
from jax.experimental import pallas as pl
from jax.experimental.pallas import tpu as pltpu
from jax.experimental.pallas import tpu_sc as plsc

_N = 100000
_E = 1600000
_F = 8
_G = 16
_NR = _N // _G
_NPAD = 100096
_NRP = _NPAD // _G
_NC = 2
_NS = 16
_NW = _NC * _NS
_EW = _E // _NW
_CHUNK = 2000
_ZRPS = _NPAD // _NS
_BFIN = 3128


def _hi_dot(a, b):
    return lax.dot_general(a, b, (((1,), (0,)), ((), ())),
                           precision=lax.Precision.HIGHEST,
                           preferred_element_type=jnp.float32)


def _tc_build_table(xt3, zt3):
    def body(x_ref, z_ref, o_ref):
        xc = jnp.concatenate([x_ref[0], x_ref[1], x_ref[2],
                              z_ref[0], z_ref[1], z_ref[2]], axis=1)
        r96 = lax.broadcasted_iota(jnp.int32, (96, 128), 0)
        l96 = lax.broadcasted_iota(jnp.int32, (96, 128), 1)
        e = ((l96 // 8 == r96 % 16) & (l96 % 8 == r96 // 16))
        lane = lax.broadcasted_iota(jnp.int32, (_BFIN, 128), 1)
        o_ref[...] = (_hi_dot(xc, e.astype(jnp.float32))
                      + (lane % 8 == 6).astype(jnp.float32))

    return pl.pallas_call(
        body,
        grid=(_NRP // _BFIN,),
        in_specs=[pl.BlockSpec((3, _BFIN, _G), lambda i: (0, i, 0)),
                  pl.BlockSpec((3, _BFIN, _G), lambda i: (0, i, 0))],
        out_specs=pl.BlockSpec((_BFIN, 128), lambda i: (i, 0)),
        out_shape=jax.ShapeDtypeStruct((_NRP, 128), jnp.float32),
    )(xt3, zt3)


def _sc_segment_sums(tab, row, col, zeros):
    mesh = plsc.VectorSubcoreMesh(core_axis_name="c", subcore_axis_name="s")

    @pl.kernel(
        out_type=jax.ShapeDtypeStruct((_NC, _NPAD, _F), jnp.float32),
        mesh=mesh,
        scratch_types=[
            pltpu.VMEM_SHARED((_NPAD, _F), jnp.float32),
            pltpu.VMEM((_CHUNK,), jnp.int32),
            pltpu.VMEM((_CHUNK,), jnp.int32),
            pltpu.VMEM((_CHUNK, _F), jnp.float32),
        ],
        compiler_params=pltpu.CompilerParams(use_tc_tiling_on_sc=False),
    )
    def kern(tab_hbm, row_hbm, col_hbm, zeros_hbm, out_hbm, acc,
             rbuf, cbuf, vals):
        c = lax.axis_index("c")
        s = lax.axis_index("s")
        wid = c * _NS + s
        nbase = s * _ZRPS

        pltpu.sync_copy(zeros_hbm.at[pl.ds(nbase, _ZRPS)],
                        acc.at[pl.ds(nbase, _ZRPS)])
        plsc.subcore_barrier()

        @pl.loop(0, _EW, step=_CHUNK)
        def _(i):
            base = wid * _EW + i
            pltpu.sync_copy(col_hbm.at[pl.ds(base, _CHUNK)], cbuf)
            pltpu.sync_copy(row_hbm.at[pl.ds(base, _CHUNK)], rbuf)
            pltpu.sync_copy(tab_hbm.at[cbuf], vals)
            pltpu.sync_copy(vals, acc.at[rbuf], add=True)

        plsc.subcore_barrier()
        pltpu.sync_copy(acc.at[pl.ds(nbase, _ZRPS)],
                        out_hbm.at[c].at[pl.ds(nbase, _ZRPS)])

    return kern(tab, row, col, zeros)


def _tc_finish(partials, tabi):
    def body(p_ref, t_ref, o_ref):
        p = p_ref[0] + p_ref[1]
        t = t_ref[...]

        lane = lax.broadcasted_iota(jnp.int32, (_BFIN, 128), 1)
        f = lane % 8
        at0 = f == 0
        at3 = f == 3

        def grp_bcast(head):
            return head + pltpu.roll(head, 1, 1) + pltpu.roll(head, 2, 1)

        chead = jnp.where(at0, pltpu.roll(p, 122, 1),
                          jnp.where(at3, pltpu.roll(p, 125, 1), 0.0))
        cnt = jnp.maximum(grp_bcast(chead), 1.0)
        mean = p / cnt
        sq = mean * mean
        s3 = sq + pltpu.roll(sq, 127, 1) + pltpu.roll(sq, 126, 1)
        nhead = jnp.where(at0 | at3, s3, 0.0)
        dir_ = mean * jax.lax.rsqrt(grp_bcast(nhead))
        d1 = jnp.abs(t - dir_)
        zdx = pltpu.roll(d1, 125, 1)
        rowi = (pl.program_id(0) * _BFIN
                + lax.broadcasted_iota(jnp.int32, (_BFIN, 128), 0))
        dif = jnp.where((f < 3) & (rowi < _NR), jnp.abs(d1 - zdx), 0.0)
        part = jnp.sum(dif) * (100.0 / (_N * 3))

        @pl.when(pl.program_id(0) == 0)
        def _():
            o_ref[0, 0] = 0.0

        o_ref[0, 0] += part

    out = pl.pallas_call(
        body,
        grid=(_NRP // _BFIN,),
        in_specs=[pl.BlockSpec((_NC, _BFIN, 128), lambda i: (0, i, 0)),
                  pl.BlockSpec((_BFIN, 128), lambda i: (i, 0))],
        out_specs=pl.BlockSpec((1, 1), lambda i: (0, 0),
                               memory_space=pltpu.SMEM),
        out_shape=jax.ShapeDtypeStruct((1, 1), jnp.float32),
    )(partials, tabi)
    return out[0, 0]


def kernel(x, row, col, xyz):
    zeros = jnp.zeros((_NPAD, _F), jnp.float32)
    xt3 = jnp.reshape(jnp.pad(x.T, ((0, 0), (0, _NPAD - _N))), (3, _NRP, _G))
    zt3 = jnp.reshape(jnp.pad(xyz.T, ((0, 0), (0, _NPAD - _N))), (3, _NRP, _G))
    tabi = _tc_build_table(xt3, zt3)
    tab = jnp.reshape(tabi, (_NPAD, _F))
    partials = _sc_segment_sums(tab, row, col, zeros)
    pint = jnp.reshape(partials, (_NC, _NRP, 128))
    return _tc_finish(pint, tabi)
